# Initial kernel scaffold; baseline (speedup 1.0000x reference)
import jax
import jax.numpy as jnp
from jax.experimental import pallas as pl


def kernel(tokenized_text, dynamic_bools, token_embedding, ca_vectors, da_vectors):
    raise NotImplementedError("write your pallas kernel here")


# speedup vs baseline: 1.4166x; 1.4166x over previous
import jax
import jax.numpy as jnp
from jax import lax
from jax.experimental import pallas as pl
from jax.experimental.pallas import tpu as pltpu
from jax.experimental.pallas import tpu_sc as plsc

D = 768
B = 1024
SEQ = 77
CA = 16
PRE = 1 + CA
REST = SEQ - PRE

_info = plsc.get_sparse_core_info()
_NC, _NS = _info.num_cores, _info.num_subcores
NW = _NC * _NS
NB = B // NW


def _body(tok0_hbm, tok1_hbm, emb_hbm, ca_hbm, out_hbm,
          tok0_v, tok1_v, cbuf, gbuf, sem0, sem1):
    wid = lax.axis_index("s") * _NC + lax.axis_index("c")
    base = wid * NB
    pltpu.sync_copy(tok0_hbm.at[pl.ds(base, NB)], tok0_v)
    pltpu.sync_copy(tok1_hbm.at[pl.ds(base, NB)], tok1_v)
    pltpu.sync_copy(ca_hbm, cbuf.at[pl.ds(1, CA)])

    def step(j, carry):
        b = base + j
        pltpu.async_copy(emb_hbm.at[tok0_v.at[j]], cbuf.at[pl.ds(0, 1)], sem0).wait()
        pltpu.async_copy(emb_hbm.at[tok1_v.at[j]], gbuf, sem1).wait()
        pltpu.sync_copy(cbuf, out_hbm.at[b, pl.ds(0, PRE)])
        pltpu.sync_copy(gbuf, out_hbm.at[b, pl.ds(PRE, REST)])
        return carry

    lax.fori_loop(0, NB, step, 0)


def kernel(tokenized_text, dynamic_bools, token_embedding, ca_vectors, da_vectors):
    tok = tokenized_text.astype(jnp.int32)
    tok0 = tok[:, :1]
    tok1 = tok[:, 1:1 + REST]
    mesh = plsc.VectorSubcoreMesh(core_axis_name="c", subcore_axis_name="s")
    f = pl.kernel(
        _body,
        mesh=mesh,
        compiler_params=pltpu.CompilerParams(use_tc_tiling_on_sc=False),
        out_type=jax.ShapeDtypeStruct((B, SEQ, D), jnp.float32),
        scratch_types=[
            pltpu.VMEM((NB, 1), jnp.int32),
            pltpu.VMEM((NB, REST), jnp.int32),
            pltpu.VMEM((PRE, D), jnp.float32),
            pltpu.VMEM((REST, D), jnp.float32),
            pltpu.SemaphoreType.DMA,
            pltpu.SemaphoreType.DMA,
        ],
    )
    return f(tok0, tok1, token_embedding, ca_vectors)


# speedup vs baseline: 1.4906x; 1.0523x over previous
import jax
import jax.numpy as jnp
from jax import lax
from jax.experimental import pallas as pl
from jax.experimental.pallas import tpu as pltpu
from jax.experimental.pallas import tpu_sc as plsc

D = 768
B = 1024
SEQ = 77
CA = 16
PRE = 1 + CA
REST = SEQ - PRE
NTOK = 1 + REST

_info = plsc.get_sparse_core_info()
_NC, _NS = _info.num_cores, _info.num_subcores
NW = _NC * _NS
NB = B // NW


def _body(tok_hbm, emb_hbm, ca_hbm, out_hbm,
          tok_v, cab, g0, g1, sg0, sg1, sw):
    wid = lax.axis_index("s") * _NC + lax.axis_index("c")
    base = wid * NB
    pltpu.sync_copy(tok_hbm.at[pl.ds(base, NB)], tok_v)
    pltpu.sync_copy(ca_hbm, cab)

    pltpu.async_copy(emb_hbm.at[tok_v.at[0]], g0, sg0)
    pltpu.async_copy(emb_hbm.at[tok_v.at[1]], g1, sg1)

    def pair(i, carry):
        g = 2 * i
        for slot, gbuf, sg in ((0, g0, sg0), (1, g1, sg1)):
            j = g + slot
            b = base + j
            w_ca = pltpu.make_async_copy(cab, out_hbm.at[b, pl.ds(1, CA)], sw)
            w_ca.start()
            pltpu.make_async_copy(emb_hbm.at[pl.ds(0, NTOK)], gbuf, sg).wait()
            w0 = pltpu.make_async_copy(
                gbuf.at[pl.ds(0, 1)], out_hbm.at[b, pl.ds(0, 1)], sw)
            w1 = pltpu.make_async_copy(
                gbuf.at[pl.ds(1, REST)], out_hbm.at[b, pl.ds(PRE, REST)], sw)
            w0.start()
            w1.start()
            w_ca.wait()
            w0.wait()
            w1.wait()

            @pl.when(j + 2 < NB)
            def _start_next():
                pltpu.async_copy(emb_hbm.at[tok_v.at[j + 2]], gbuf, sg)

        return carry

    lax.fori_loop(0, NB // 2, pair, 0)


def kernel(tokenized_text, dynamic_bools, token_embedding, ca_vectors, da_vectors):
    tok = tokenized_text.astype(jnp.int32)[:, :NTOK]
    mesh = plsc.VectorSubcoreMesh(core_axis_name="c", subcore_axis_name="s")
    f = pl.kernel(
        _body,
        mesh=mesh,
        compiler_params=pltpu.CompilerParams(use_tc_tiling_on_sc=False),
        out_type=jax.ShapeDtypeStruct((B, SEQ, D), jnp.float32),
        scratch_types=[
            pltpu.VMEM((NB, NTOK), jnp.int32),
            pltpu.VMEM((CA, D), jnp.float32),
            pltpu.VMEM((NTOK, D), jnp.float32),
            pltpu.VMEM((NTOK, D), jnp.float32),
            pltpu.SemaphoreType.DMA,
            pltpu.SemaphoreType.DMA,
            pltpu.SemaphoreType.DMA,
        ],
    )
    return f(tok, token_embedding, ca_vectors)
